# baseline (speedup 1.0000x reference)
import functools

import jax
import jax.numpy as jnp
from jax import lax
from jax.experimental import pallas as pl
from jax.experimental.pallas import tpu as pltpu
from jax.experimental.pallas import tpu_sc as plsc

NC = 2
NS = 16
NW = NC * NS
LANES = 16

BATCH = 4
SEQ = 8192
DIM = 1024
ROWS_W = SEQ // NW
T = 8
CHUNKS = ROWS_W // T


def _sc_body(x_hbm, pos_hbm, out_hbm,
             xb0, xb1, pb0, pb1,
             xi0, xi1, xo0, xo1, ps0, ps1):
    xb = [xb0, xb1]
    pb = [pb0, pb1]
    xisem = [xi0, xi1]
    xosem = [xo0, xo1]
    psem = [ps0, ps1]

    wid = lax.axis_index("s") * NC + lax.axis_index("c")
    base = wid * ROWS_W
    last_ci = CHUNKS - 1

    def issue_x_in(ci, q):
        pltpu.async_copy(
            x_hbm.at[:, pl.ds(base + ci * T, T), :], xb[q], xisem[q])

    def issue_x_out(ci, q):
        pltpu.async_copy(
            xb[q], out_hbm.at[:, pl.ds(base + ci * T, T), :], xosem[q])

    def issue_pos(ci, q):
        pltpu.async_copy(
            pos_hbm.at[pl.ds(base + ci * T, T)], pb[q], psem[q])

    def wait_x_in(q):
        pltpu.make_async_copy(
            x_hbm.at[:, pl.ds(0, T), :], xb[q], xisem[q]).wait()

    def wait_x_out(q):
        pltpu.make_async_copy(
            xb[q], out_hbm.at[:, pl.ds(0, T), :], xosem[q]).wait()

    def wait_pos(q):
        pltpu.make_async_copy(
            pos_hbm.at[pl.ds(0, T)], pb[q], psem[q]).wait()

    issue_pos(0, 0)
    issue_x_in(0, 0)

    @pl.loop(0, CHUNKS, step=2)
    def _(ci0):
        for q in (0, 1):
            ci = ci0 + q
            ci_next = jnp.minimum(ci + 1, last_ci)
            wait_pos(q)
            issue_pos(ci_next, 1 - q)
            if q == 0:
                @pl.when(ci0 > 0)
                def _():
                    wait_x_out(1)
            else:
                wait_x_out(0)
            issue_x_in(ci_next, 1 - q)
            wait_x_in(q)
            pbuf = pb[q]
            xbuf = xb[q]

            @plsc.parallel_loop(0, T, 1)
            def _(r):
                for j in range(DIM // LANES):
                    c = j * LANES
                    pv = pbuf[r, pl.ds(c, LANES)]
                    for b in range(BATCH):
                        xbuf[b, r, pl.ds(c, LANES)] = (
                            xbuf[b, r, pl.ds(c, LANES)] + pv
                        )

            issue_x_out(ci, q)

    wait_x_out((CHUNKS - 1) % 2)
    wait_x_in(CHUNKS % 2)
    wait_pos(CHUNKS % 2)


_sc_call = functools.partial(
    pl.kernel,
    out_type=jax.ShapeDtypeStruct((BATCH, SEQ, DIM), jnp.float32),
    mesh=plsc.VectorSubcoreMesh(core_axis_name="c", subcore_axis_name="s"),
    scratch_types=[
        pltpu.VMEM((BATCH, T, DIM), jnp.float32),
        pltpu.VMEM((BATCH, T, DIM), jnp.float32),
        pltpu.VMEM((T, DIM), jnp.float32),
        pltpu.VMEM((T, DIM), jnp.float32),
        pltpu.SemaphoreType.DMA,
        pltpu.SemaphoreType.DMA,
        pltpu.SemaphoreType.DMA,
        pltpu.SemaphoreType.DMA,
        pltpu.SemaphoreType.DMA,
        pltpu.SemaphoreType.DMA,
    ],
)(_sc_body)


def kernel(x, pos_table):
    batch, seq_len, dim = x.shape
    return _sc_call(x, pos_table[:seq_len])
